# baseline (speedup 1.0000x reference)
import jax
import jax.numpy as jnp
from jax.experimental import pallas as pl


def _add_kernel(x_ref, pe_ref, o_ref):
    o_ref[...] = x_ref[...] + pe_ref[...]


def kernel(x, pe_weight):
    B, S, D = x.shape
    SB = 512
    n_pe = S // SB
    xf = x.reshape(B * S, D)
    out = pl.pallas_call(
        _add_kernel,
        grid=(B * S // SB,),
        in_specs=[
            pl.BlockSpec((SB, D), lambda i: (i, 0)),
            pl.BlockSpec((SB, D), lambda i: (i % n_pe, 0)),
        ],
        out_specs=pl.BlockSpec((SB, D), lambda i: (i, 0)),
        out_shape=jax.ShapeDtypeStruct((B * S, D), x.dtype),
    )(xf, pe_weight)
    return out.reshape(B, S, D)
